# Initial kernel scaffold; baseline (speedup 1.0000x reference)
import jax
import jax.numpy as jnp
from jax.experimental import pallas as pl


def kernel(x, edge_index, W0, b0, W1, b1, W2, b2, W_out, b_out):
    raise NotImplementedError("write your pallas kernel here")


# speedup vs baseline: 4.9748x; 4.9748x over previous
import functools

import jax
import jax.numpy as jnp
from jax import lax
from jax.experimental import pallas as pl
from jax.experimental.pallas import tpu as pltpu
from jax.experimental.pallas import tpu_sc as plsc

N = 10000
E = 320000
H = 128

NC = 2
NS = 16
NW = NC * NS
EPW = E // NW
CHUNK = 80
NCHUNK = EPW // CHUNK
NP = 10240
RPT = NP // NS

_MESH = plsc.VectorSubcoreMesh(core_axis_name="c", subcore_axis_name="s")


def _scatter_body(h_hbm, src_hbm, dst_hbm, zeros_hbm, out0_hbm, out1_hbm,
                  acc, idx_src, idx_dst, rows, sem):
    c = lax.axis_index("c")
    s = lax.axis_index("s")
    wid = c * NS + s

    row0 = s * RPT
    pltpu.sync_copy(zeros_hbm, acc.at[pl.ds(row0, RPT)])
    plsc.subcore_barrier()

    @pl.loop(0, NCHUNK)
    def _chunk(j):
        base = wid * EPW + j * CHUNK
        pltpu.sync_copy(src_hbm.at[pl.ds(base, CHUNK)], idx_src)
        pltpu.sync_copy(dst_hbm.at[pl.ds(base, CHUNK)], idx_dst)
        pltpu.async_copy(h_hbm.at[idx_src], rows, sem).wait()
        pltpu.sync_copy(rows, acc.at[idx_dst], add=True)

    plsc.subcore_barrier()

    @pl.when(c == 0)
    def _():
        pltpu.sync_copy(acc.at[pl.ds(row0, RPT)], out0_hbm.at[pl.ds(row0, RPT)])

    @pl.when(c == 1)
    def _():
        pltpu.sync_copy(acc.at[pl.ds(row0, RPT)], out1_hbm.at[pl.ds(row0, RPT)])


_scatter = pl.kernel(
    _scatter_body,
    out_type=(jax.ShapeDtypeStruct((NP, H), jnp.float32),
              jax.ShapeDtypeStruct((NP, H), jnp.float32)),
    mesh=_MESH,
    scratch_types=[
        pltpu.VMEM_SHARED((NP, H), jnp.float32),
        pltpu.VMEM((CHUNK,), jnp.int32),
        pltpu.VMEM((CHUNK,), jnp.int32),
        pltpu.VMEM((CHUNK, H), jnp.float32),
        pltpu.SemaphoreType.DMA,
    ],
)


BM = 1000


def _layer_tc(p0_ref, p1_ref, w_ref, b_ref, h_ref):
    m = p0_ref[...] + p1_ref[...]
    y = jnp.dot(m, w_ref[...], preferred_element_type=jnp.float32) + b_ref[...]
    h_ref[...] = jnp.where(y > 0, y, 0.1 * y)


def _layer_call(p0, p1, W, b):
    return pl.pallas_call(
        _layer_tc,
        grid=(N // BM,),
        in_specs=[
            pl.BlockSpec((BM, H), lambda i: (i, 0)),
            pl.BlockSpec((BM, H), lambda i: (i, 0)),
            pl.BlockSpec((H, H), lambda i: (0, 0)),
            pl.BlockSpec((1, H), lambda i: (0, 0)),
        ],
        out_specs=pl.BlockSpec((BM, H), lambda i: (i, 0)),
        out_shape=jax.ShapeDtypeStruct((N, H), jnp.float32),
    )(p0, p1, W, b.reshape(1, H))


def _final_tc(p0_ref, p1_ref, w2_ref, b2_ref, x_ref, h1_ref, h2_ref,
              wo_ref, bo_ref, out_ref):
    m = p0_ref[...] + p1_ref[...]
    y = jnp.dot(m, w2_ref[...], preferred_element_type=jnp.float32) + b2_ref[...]
    h3 = jnp.where(y > 0, y, 0.1 * y)
    o = (jnp.dot(x_ref[...], wo_ref[0], preferred_element_type=jnp.float32)
         + jnp.dot(h1_ref[...], wo_ref[1], preferred_element_type=jnp.float32)
         + jnp.dot(h2_ref[...], wo_ref[2], preferred_element_type=jnp.float32)
         + jnp.dot(h3, wo_ref[3], preferred_element_type=jnp.float32)
         + bo_ref[...])
    out_ref[...] = jnp.where(o > 0, o, 0.1 * o)


def _final_call(p0, p1, W2, b2, x, h1, h2, W_out, b_out):
    return pl.pallas_call(
        _final_tc,
        grid=(N // BM,),
        in_specs=[
            pl.BlockSpec((BM, H), lambda i: (i, 0)),
            pl.BlockSpec((BM, H), lambda i: (i, 0)),
            pl.BlockSpec((H, H), lambda i: (0, 0)),
            pl.BlockSpec((1, H), lambda i: (0, 0)),
            pl.BlockSpec((BM, H), lambda i: (i, 0)),
            pl.BlockSpec((BM, H), lambda i: (i, 0)),
            pl.BlockSpec((BM, H), lambda i: (i, 0)),
            pl.BlockSpec((4, H, H), lambda i: (0, 0, 0)),
            pl.BlockSpec((1, H), lambda i: (0, 0)),
        ],
        out_specs=pl.BlockSpec((BM, H), lambda i: (i, 0)),
        out_shape=jax.ShapeDtypeStruct((N, H), jnp.float32),
    )(p0, p1, W2, b2.reshape(1, H), x, h1, h2,
      W_out.reshape(4, H, H), b_out.reshape(1, H))


def kernel(x, edge_index, W0, b0, W1, b1, W2, b2, W_out, b_out):
    src = edge_index[0]
    dst = edge_index[1]
    zeros = jnp.zeros((RPT, H), jnp.float32)

    p0a, p0b = _scatter(x, src, dst, zeros)
    h1 = _layer_call(p0a, p0b, W0, b0)
    p1a, p1b = _scatter(h1, src, dst, zeros)
    h2 = _layer_call(p1a, p1b, W1, b1)
    p2a, p2b = _scatter(h2, src, dst, zeros)
    return _final_call(p2a, p2b, W2, b2, x, h1, h2, W_out, b_out)


# speedup vs baseline: 14.1211x; 2.8385x over previous
import functools

import jax
import jax.numpy as jnp
from jax import lax
from jax.experimental import pallas as pl
from jax.experimental.pallas import tpu as pltpu
from jax.experimental.pallas import tpu_sc as plsc

N = 10000
E = 320000
H = 128

NC = 2
NS = 16
NW = NC * NS
EPW = E // NW
CHUNK = 40
NCHUNK = EPW // CHUNK
NP = 10240
RPT = NP // NS

_MESH = plsc.VectorSubcoreMesh(core_axis_name="c", subcore_axis_name="s")


NBUF = 5


def _scatter_body(h_hbm, src_hbm, dst_hbm, zeros_hbm, out0_hbm, out1_hbm,
                  acc, srcv, dstv, *rest):
    rows = rest[:NBUF]
    dsts = rest[NBUF:2 * NBUF]
    sems = rest[2 * NBUF:]
    c = lax.axis_index("c")
    s = lax.axis_index("s")
    wid = c * NS + s

    row0 = s * RPT
    pltpu.sync_copy(zeros_hbm, acc.at[pl.ds(row0, RPT)])
    pltpu.sync_copy(src_hbm.at[pl.ds(wid * EPW, EPW)], srcv)
    pltpu.sync_copy(dst_hbm.at[pl.ds(wid * EPW, EPW)], dstv)

    for b in range(NBUF):
        pltpu.async_copy(h_hbm.at[srcv.at[pl.ds(b * CHUNK, CHUNK)]],
                         rows[b], sems[b])
    plsc.subcore_barrier()

    @pl.loop(0, NCHUNK // NBUF)
    def _group(jj):
        for b in range(NBUF):
            j = jj * NBUF + b
            pltpu.make_async_copy(h_hbm.at[pl.ds(0, CHUNK)], rows[b],
                                  sems[b]).wait()
            pltpu.sync_copy(rows[b], acc.at[dstv.at[pl.ds(j * CHUNK, CHUNK)]],
                            add=True)
            nxt = j + NBUF

            @pl.when(nxt < NCHUNK)
            def _():
                pltpu.async_copy(
                    h_hbm.at[srcv.at[pl.ds(nxt * CHUNK, CHUNK)]],
                    rows[b], sems[b])

    plsc.subcore_barrier()

    @pl.when(c == 0)
    def _():
        pltpu.sync_copy(acc.at[pl.ds(row0, RPT)], out0_hbm.at[pl.ds(row0, RPT)])

    @pl.when(c == 1)
    def _():
        pltpu.sync_copy(acc.at[pl.ds(row0, RPT)], out1_hbm.at[pl.ds(row0, RPT)])


_scatter = pl.kernel(
    _scatter_body,
    out_type=(jax.ShapeDtypeStruct((NP, H), jnp.float32),
              jax.ShapeDtypeStruct((NP, H), jnp.float32)),
    mesh=_MESH,
    scratch_types=[
        pltpu.VMEM_SHARED((NP, H), jnp.float32),
        pltpu.VMEM((EPW,), jnp.int32),
        pltpu.VMEM((EPW,), jnp.int32),
    ] + [pltpu.VMEM((CHUNK, H), jnp.float32) for _ in range(NBUF)]
      + [pltpu.VMEM((CHUNK,), jnp.int32) for _ in range(NBUF)]
      + [pltpu.SemaphoreType.DMA for _ in range(NBUF)],
)


BM = 1000


def _layer_tc(p0_ref, p1_ref, w_ref, b_ref, h_ref):
    m = p0_ref[...] + p1_ref[...]
    y = jnp.dot(m, w_ref[...], preferred_element_type=jnp.float32) + b_ref[...]
    h_ref[...] = jnp.where(y > 0, y, 0.1 * y)


def _layer_call(p0, p1, W, b):
    return pl.pallas_call(
        _layer_tc,
        grid=(N // BM,),
        in_specs=[
            pl.BlockSpec((BM, H), lambda i: (i, 0)),
            pl.BlockSpec((BM, H), lambda i: (i, 0)),
            pl.BlockSpec((H, H), lambda i: (0, 0)),
            pl.BlockSpec((1, H), lambda i: (0, 0)),
        ],
        out_specs=pl.BlockSpec((BM, H), lambda i: (i, 0)),
        out_shape=jax.ShapeDtypeStruct((N, H), jnp.float32),
    )(p0, p1, W, b.reshape(1, H))


def _final_tc(p0_ref, p1_ref, w2_ref, b2_ref, x_ref, h1_ref, h2_ref,
              wo_ref, bo_ref, out_ref):
    m = p0_ref[...] + p1_ref[...]
    y = jnp.dot(m, w2_ref[...], preferred_element_type=jnp.float32) + b2_ref[...]
    h3 = jnp.where(y > 0, y, 0.1 * y)
    o = (jnp.dot(x_ref[...], wo_ref[0], preferred_element_type=jnp.float32)
         + jnp.dot(h1_ref[...], wo_ref[1], preferred_element_type=jnp.float32)
         + jnp.dot(h2_ref[...], wo_ref[2], preferred_element_type=jnp.float32)
         + jnp.dot(h3, wo_ref[3], preferred_element_type=jnp.float32)
         + bo_ref[...])
    out_ref[...] = jnp.where(o > 0, o, 0.1 * o)


def _final_call(p0, p1, W2, b2, x, h1, h2, W_out, b_out):
    return pl.pallas_call(
        _final_tc,
        grid=(N // BM,),
        in_specs=[
            pl.BlockSpec((BM, H), lambda i: (i, 0)),
            pl.BlockSpec((BM, H), lambda i: (i, 0)),
            pl.BlockSpec((H, H), lambda i: (0, 0)),
            pl.BlockSpec((1, H), lambda i: (0, 0)),
            pl.BlockSpec((BM, H), lambda i: (i, 0)),
            pl.BlockSpec((BM, H), lambda i: (i, 0)),
            pl.BlockSpec((BM, H), lambda i: (i, 0)),
            pl.BlockSpec((4, H, H), lambda i: (0, 0, 0)),
            pl.BlockSpec((1, H), lambda i: (0, 0)),
        ],
        out_specs=pl.BlockSpec((BM, H), lambda i: (i, 0)),
        out_shape=jax.ShapeDtypeStruct((N, H), jnp.float32),
    )(p0, p1, W2, b2.reshape(1, H), x, h1, h2,
      W_out.reshape(4, H, H), b_out.reshape(1, H))


def kernel(x, edge_index, W0, b0, W1, b1, W2, b2, W_out, b_out):
    src = edge_index[0]
    dst = edge_index[1]
    zeros = jnp.zeros((RPT, H), jnp.float32)

    p0a, p0b = _scatter(x, src, dst, zeros)
    h1 = _layer_call(p0a, p0b, W0, b0)
    p1a, p1b = _scatter(h1, src, dst, zeros)
    h2 = _layer_call(p1a, p1b, W1, b1)
    p2a, p2b = _scatter(h2, src, dst, zeros)
    return _final_call(p2a, p2b, W2, b2, x, h1, h2, W_out, b_out)


# speedup vs baseline: 14.2194x; 1.0070x over previous
import functools

import jax
import jax.numpy as jnp
from jax import lax
from jax.experimental import pallas as pl
from jax.experimental.pallas import tpu as pltpu
from jax.experimental.pallas import tpu_sc as plsc

N = 10000
E = 320000
H = 128

NC = 2
NS = 16
NW = NC * NS
EPW = E // NW
CHUNK = 40
NCHUNK = EPW // CHUNK
NP = 10240
RPT = NP // NS

_MESH = plsc.VectorSubcoreMesh(core_axis_name="c", subcore_axis_name="s")


NBUF = 5


def _scatter_body(h_hbm, src_hbm, dst_hbm, zeros_hbm, out0_hbm, out1_hbm,
                  acc, srcv, dstv, *rest):
    rows = rest[:NBUF]
    gsem = rest[NBUF:2 * NBUF]
    ssem = rest[2 * NBUF:]
    c = lax.axis_index("c")
    s = lax.axis_index("s")
    wid = c * NS + s

    row0 = s * RPT
    pltpu.sync_copy(zeros_hbm, acc.at[pl.ds(row0, RPT)])
    pltpu.sync_copy(src_hbm.at[pl.ds(wid * EPW, EPW)], srcv)
    pltpu.sync_copy(dst_hbm.at[pl.ds(wid * EPW, EPW)], dstv)

    def _scatter_desc(k, b):
        return pltpu.make_async_copy(
            rows[b], acc.at[dstv.at[pl.ds(k * CHUNK, CHUNK)]], ssem[b])

    for b in range(NBUF - 1):
        pltpu.async_copy(h_hbm.at[srcv.at[pl.ds(b * CHUNK, CHUNK)]],
                         rows[b], gsem[b])
    plsc.subcore_barrier()

    @pl.loop(0, NCHUNK // NBUF)
    def _group(jj):
        for b in range(NBUF):
            k = jj * NBUF + b
            b2 = (b - 1) % NBUF
            pltpu.make_async_copy(h_hbm.at[pl.ds(0, CHUNK)], rows[b],
                                  gsem[b]).wait()

            @pl.when(k > 0)
            def _():
                _scatter_desc(k - 1, b2).wait()

            m = k + NBUF - 1

            @pl.when(m < NCHUNK)
            def _():
                pltpu.async_copy(
                    h_hbm.at[srcv.at[pl.ds(m * CHUNK, CHUNK)]],
                    rows[b2], gsem[b2])

            pltpu.async_copy(rows[b],
                             acc.at[dstv.at[pl.ds(k * CHUNK, CHUNK)]],
                             ssem[b], add=True)

    _scatter_desc(NCHUNK - 1, (NCHUNK - 1) % NBUF).wait()
    plsc.subcore_barrier()

    @pl.when(c == 0)
    def _():
        pltpu.sync_copy(acc.at[pl.ds(row0, RPT)], out0_hbm.at[pl.ds(row0, RPT)])

    @pl.when(c == 1)
    def _():
        pltpu.sync_copy(acc.at[pl.ds(row0, RPT)], out1_hbm.at[pl.ds(row0, RPT)])


_scatter = pl.kernel(
    _scatter_body,
    out_type=(jax.ShapeDtypeStruct((NP, H), jnp.float32),
              jax.ShapeDtypeStruct((NP, H), jnp.float32)),
    mesh=_MESH,
    scratch_types=[
        pltpu.VMEM_SHARED((NP, H), jnp.float32),
        pltpu.VMEM((EPW,), jnp.int32),
        pltpu.VMEM((EPW,), jnp.int32),
    ] + [pltpu.VMEM((CHUNK, H), jnp.float32) for _ in range(NBUF)]
      + [pltpu.SemaphoreType.DMA for _ in range(2 * NBUF)],
)


BM = 1000


def _layer_tc(p0_ref, p1_ref, w_ref, b_ref, h_ref):
    m = p0_ref[...] + p1_ref[...]
    y = jnp.dot(m, w_ref[...], preferred_element_type=jnp.float32) + b_ref[...]
    h_ref[...] = jnp.where(y > 0, y, 0.1 * y)


def _layer_call(p0, p1, W, b):
    return pl.pallas_call(
        _layer_tc,
        grid=(N // BM,),
        in_specs=[
            pl.BlockSpec((BM, H), lambda i: (i, 0)),
            pl.BlockSpec((BM, H), lambda i: (i, 0)),
            pl.BlockSpec((H, H), lambda i: (0, 0)),
            pl.BlockSpec((1, H), lambda i: (0, 0)),
        ],
        out_specs=pl.BlockSpec((BM, H), lambda i: (i, 0)),
        out_shape=jax.ShapeDtypeStruct((N, H), jnp.float32),
    )(p0, p1, W, b.reshape(1, H))


def _final_tc(p0_ref, p1_ref, w2_ref, b2_ref, x_ref, h1_ref, h2_ref,
              wo_ref, bo_ref, out_ref):
    m = p0_ref[...] + p1_ref[...]
    y = jnp.dot(m, w2_ref[...], preferred_element_type=jnp.float32) + b2_ref[...]
    h3 = jnp.where(y > 0, y, 0.1 * y)
    o = (jnp.dot(x_ref[...], wo_ref[0], preferred_element_type=jnp.float32)
         + jnp.dot(h1_ref[...], wo_ref[1], preferred_element_type=jnp.float32)
         + jnp.dot(h2_ref[...], wo_ref[2], preferred_element_type=jnp.float32)
         + jnp.dot(h3, wo_ref[3], preferred_element_type=jnp.float32)
         + bo_ref[...])
    out_ref[...] = jnp.where(o > 0, o, 0.1 * o)


def _final_call(p0, p1, W2, b2, x, h1, h2, W_out, b_out):
    return pl.pallas_call(
        _final_tc,
        grid=(N // BM,),
        in_specs=[
            pl.BlockSpec((BM, H), lambda i: (i, 0)),
            pl.BlockSpec((BM, H), lambda i: (i, 0)),
            pl.BlockSpec((H, H), lambda i: (0, 0)),
            pl.BlockSpec((1, H), lambda i: (0, 0)),
            pl.BlockSpec((BM, H), lambda i: (i, 0)),
            pl.BlockSpec((BM, H), lambda i: (i, 0)),
            pl.BlockSpec((BM, H), lambda i: (i, 0)),
            pl.BlockSpec((4, H, H), lambda i: (0, 0, 0)),
            pl.BlockSpec((1, H), lambda i: (0, 0)),
        ],
        out_specs=pl.BlockSpec((BM, H), lambda i: (i, 0)),
        out_shape=jax.ShapeDtypeStruct((N, H), jnp.float32),
    )(p0, p1, W2, b2.reshape(1, H), x, h1, h2,
      W_out.reshape(4, H, H), b_out.reshape(1, H))


def kernel(x, edge_index, W0, b0, W1, b1, W2, b2, W_out, b_out):
    src = edge_index[0]
    dst = edge_index[1]
    zeros = jnp.zeros((RPT, H), jnp.float32)

    p0a, p0b = _scatter(x, src, dst, zeros)
    h1 = _layer_call(p0a, p0b, W0, b0)
    p1a, p1b = _scatter(h1, src, dst, zeros)
    h2 = _layer_call(p1a, p1b, W1, b1)
    p2a, p2b = _scatter(h2, src, dst, zeros)
    return _final_call(p2a, p2b, W2, b2, x, h1, h2, W_out, b_out)
